# baseline (speedup 1.0000x reference)
import functools

import jax
import jax.numpy as jnp
from jax import lax
from jax.experimental import pallas as pl
from jax.experimental.pallas import tpu as pltpu
from jax.experimental.pallas import tpu_sc as plsc

NUM_CORES = 2
NUM_SUBCORES = 16
NUM_WORKERS = NUM_CORES * NUM_SUBCORES
GROUP = 256
LANES = 16


def kernel(indices, table):
    batch, n_fields = indices.shape
    n_rows, dim = table.shape
    total = batch * n_fields
    pack = 128 // dim
    assert total % (NUM_WORKERS * GROUP) == 0
    per_worker = total // NUM_WORKERS
    n_groups = per_worker // GROUP
    assert n_groups % 2 == 0
    out_per_worker = per_worker // pack
    g_out = GROUP // pack

    table128 = table.reshape(n_rows // pack, dim * pack)
    idx3 = indices.reshape(NUM_WORKERS, n_groups, GROUP)
    mesh = plsc.VectorSubcoreMesh(core_axis_name="c", subcore_axis_name="s")

    @functools.partial(
        pl.kernel,
        mesh=mesh,
        out_type=jax.ShapeDtypeStruct((total // pack, dim * pack), jnp.float32),
        scratch_types=[
            pltpu.VMEM((n_groups, GROUP), jnp.int32),
            pltpu.VMEM((n_groups, GROUP), jnp.int32),
            pltpu.VMEM((per_worker,), jnp.int32),
            pltpu.VMEM((GROUP, dim * pack), jnp.float32),
            pltpu.VMEM((GROUP, dim * pack), jnp.float32),
            pltpu.VMEM((g_out, dim * pack), jnp.float32),
            pltpu.VMEM((g_out, dim * pack), jnp.float32),
            pltpu.SemaphoreType.DMA,
            pltpu.SemaphoreType.DMA,
            pltpu.SemaphoreType.DMA,
            pltpu.SemaphoreType.DMA,
        ],
        compiler_params=pltpu.CompilerParams(use_tc_tiling_on_sc=False),
    )
    def gather_kernel(idx_hbm, table_hbm, out_hbm, idx_v, pidx_v, qoff_v,
                      gbuf0, gbuf1, sbuf0, sbuf1, gsem0, gsem1, ssem0, ssem1):
        wid = lax.axis_index("s") * NUM_CORES + lax.axis_index("c")
        base = wid * out_per_worker
        pltpu.sync_copy(idx_hbm.at[wid], idx_v)

        @pl.loop(0, n_groups)
        def _(g):
            for k in range(GROUP // LANES):
                v = idx_v[g, pl.ds(k * LANES, LANES)]
                pidx_v[g, pl.ds(k * LANES, LANES)] = v >> (pack.bit_length() - 1)
                qoff_v[pl.ds(g * GROUP + k * LANES, LANES)] = (
                    (v & (pack - 1)) << (dim.bit_length() - 1)
                )

        def fire_gather(g, gbuf, gsem):
            pltpu.async_copy(table_hbm.at[pidx_v.at[g]], gbuf, gsem)

        def drain_gather(gbuf, gsem):
            pltpu.make_async_copy(
                table_hbm.at[pl.ds(0, GROUP)], gbuf, gsem
            ).wait()

        def compact(g, gbuf, sbuf):
            @pl.loop(0, GROUP // LANES)
            def _(b):
                qv = qoff_v[pl.ds(g * GROUP + b * LANES, LANES)]
                for t in range(LANES):
                    r = b * LANES + t
                    off = pl.multiple_of(qv[t], dim)
                    o = b * (LANES // pack) + t // pack
                    for h in range(dim // LANES):
                        sbuf[o, pl.ds((t % pack) * dim + h * LANES, LANES)] = (
                            gbuf[r, pl.ds(off + h * LANES, LANES)]
                        )

        def fire_store(g, sbuf, ssem):
            pltpu.async_copy(
                sbuf, out_hbm.at[pl.ds(base + g * g_out, g_out)], ssem
            )

        def drain_store(g, sbuf, ssem):
            pltpu.make_async_copy(
                sbuf, out_hbm.at[pl.ds(base + g * g_out, g_out)], ssem
            ).wait()

        fire_gather(0, gbuf0, gsem0)

        @pl.loop(0, n_groups, step=2)
        def _(i2):
            g0 = i2
            g1 = i2 + 1

            @pl.when(g0 > 0)
            def _():
                drain_store(g0 - 1, sbuf1, ssem1)

            fire_gather(g1, gbuf1, gsem1)
            drain_gather(gbuf0, gsem0)

            @pl.when(g0 > 0)
            def _():
                drain_store(g0 - 2, sbuf0, ssem0)

            compact(g0, gbuf0, sbuf0)
            fire_store(g0, sbuf0, ssem0)

            @pl.when(g1 + 1 < n_groups)
            def _():
                fire_gather(g1 + 1, gbuf0, gsem0)

            drain_gather(gbuf1, gsem1)
            compact(g1, gbuf1, sbuf1)
            fire_store(g1, sbuf1, ssem1)

        drain_store(n_groups - 2, sbuf0, ssem0)
        drain_store(n_groups - 1, sbuf1, ssem1)

    out = gather_kernel(idx3, table128)
    return out.reshape(batch, n_fields, dim)
